# baseline (speedup 1.0000x reference)
import functools

import jax
import jax.numpy as jnp
from jax import lax
from jax.experimental import pallas as pl
from jax.experimental.pallas import tpu as pltpu
from jax.experimental.pallas import tpu_sc as plsc

DIM = 32
L = 16
NC, NS = 2, 16
NW = NC * NS
CHUNK = 128
NB = 4
SCALE = 32.0 ** 0.5


def kernel(tokens, table):
    n_seq, n_batch = tokens.shape[1], tokens.shape[0]
    n_tok = n_seq * n_batch
    b_per_w = n_tok // NW
    chunks = b_per_w // CHUNK
    t_bits = 14

    idx = tokens.T.reshape(NW, chunks, CHUNK).astype(jnp.int32)
    table = table * SCALE
    mesh = plsc.VectorSubcoreMesh(core_axis_name="c", subcore_axis_name="s")

    @functools.partial(
        pl.kernel,
        out_type=jax.ShapeDtypeStruct((n_seq, DIM, n_batch), jnp.float32),
        mesh=mesh,
        scratch_types=[
            pltpu.VMEM((chunks, CHUNK), jnp.int32),
            [pltpu.VMEM((CHUNK, DIM), jnp.float32) for _ in range(NB)],
            [pltpu.VMEM((DIM, CHUNK), jnp.float32) for _ in range(NB)],
            [pltpu.SemaphoreType.DMA for _ in range(NB)],
            [pltpu.SemaphoreType.DMA for _ in range(NB)],
        ],
        compiler_params=pltpu.CompilerParams(
            use_tc_tiling_on_sc=False, needs_layout_passes=False),
    )
    def emb(table_hbm, idx_hbm, out_hbm, idx_v, g, o, gsem, ssem):
        wid = lax.axis_index("s") * NC + lax.axis_index("c")
        base = wid * b_per_w
        pltpu.sync_copy(idx_hbm.at[wid], idx_v)

        def gather(j, b):
            return pltpu.make_async_copy(
                table_hbm.at[idx_v.at[j]], g[b], gsem[b])

        def store(j, b):
            i0 = base + j * CHUNK
            s = i0 >> t_bits
            t0 = pl.multiple_of(i0 & (n_batch - 1), CHUNK)
            return pltpu.make_async_copy(
                o[b], out_hbm.at[s, :, pl.ds(t0, CHUNK)], ssem[b])

        def extract(b):
            @plsc.parallel_loop(0, DIM, unroll=2)
            def _(f):
                cf = jnp.full((L,), f, jnp.int32)
                for k in range(CHUNK // L):
                    rk = lax.iota(jnp.int32, L) + k * L
                    v = plsc.load_gather(g[b], [rk, cf])
                    o[b][f, pl.ds(k * L, L)] = v

        def process(j, b):
            gather(j, b).wait()
            extract(b)
            store(j, b).start()

        gather(0, 0).start()
        gather(1, 1).start()
        for j in (0, 1):
            gather(j + 2, (j + 2) % NB).start()
            process(j, j % NB)

        @pl.loop(0, (chunks - NB) // NB)
        def _(p):
            for b in range(NB):
                jj = 2 + p * NB + b
                bufB = b
                store(jj - 2, bufB).wait()
                gather(jj + 2, bufB).start()
                process(jj, (2 + b) % NB)

        for j in (chunks - 2, chunks - 1):
            process(j, j % NB)
        for b in range(NB):
            store(chunks - NB + b, b).wait()

    return emb(table, idx).transpose(2, 0, 1)


# speedup vs baseline: 1.3395x; 1.3395x over previous
import functools

import jax
import jax.numpy as jnp
from jax import lax
from jax.experimental import pallas as pl
from jax.experimental.pallas import tpu as pltpu
from jax.experimental.pallas import tpu_sc as plsc

DIM = 32
L = 16
NC, NS = 2, 16
NW = NC * NS
CHUNK = 128
NB = 4
SCALE = 32.0 ** 0.5


def kernel(tokens, table):
    n_seq, n_batch = tokens.shape[1], tokens.shape[0]
    n_tok = n_seq * n_batch
    b_per_w = n_tok // NW
    chunks = b_per_w // CHUNK
    t_bits = 14

    idx = tokens.T.reshape(NW, chunks, CHUNK).astype(jnp.int32)
    mesh = plsc.VectorSubcoreMesh(core_axis_name="c", subcore_axis_name="s")

    @functools.partial(
        pl.kernel,
        out_type=jax.ShapeDtypeStruct((n_seq, DIM, n_batch), jnp.float32),
        mesh=mesh,
        scratch_types=[
            pltpu.VMEM((chunks, CHUNK), jnp.int32),
            [pltpu.VMEM((CHUNK, DIM), jnp.float32) for _ in range(NB)],
            [pltpu.VMEM((DIM, CHUNK), jnp.float32) for _ in range(NB)],
            [pltpu.SemaphoreType.DMA for _ in range(NB)],
            [pltpu.SemaphoreType.DMA for _ in range(NB)],
        ],
        compiler_params=pltpu.CompilerParams(
            use_tc_tiling_on_sc=False, needs_layout_passes=False),
    )
    def emb(table_hbm, idx_hbm, out_hbm, idx_v, g, o, gsem, ssem):
        wid = lax.axis_index("s") * NC + lax.axis_index("c")
        base = wid * b_per_w
        pltpu.sync_copy(idx_hbm.at[wid], idx_v)

        def gather(j, b):
            return pltpu.make_async_copy(
                table_hbm.at[idx_v.at[j]], g[b], gsem[b])

        def store(j, b):
            i0 = base + j * CHUNK
            s = i0 >> t_bits
            t0 = pl.multiple_of(i0 & (n_batch - 1), CHUNK)
            return pltpu.make_async_copy(
                o[b], out_hbm.at[s, :, pl.ds(t0, CHUNK)], ssem[b])

        def extract(b):
            @plsc.parallel_loop(0, DIM, unroll=4)
            def _(f):
                cf = jnp.full((L,), f, jnp.int32)
                for k in range(CHUNK // L):
                    rk = lax.iota(jnp.int32, L) + k * L
                    v = plsc.load_gather(g[b], [rk, cf])
                    o[b][f, pl.ds(k * L, L)] = v * SCALE

        def process(j, b):
            gather(j, b).wait()
            extract(b)
            store(j, b).start()

        gather(0, 0).start()
        gather(1, 1).start()
        for j in (0, 1):
            gather(j + 2, (j + 2) % NB).start()
            process(j, j % NB)

        @pl.loop(0, (chunks - NB) // NB)
        def _(p):
            for b in range(NB):
                jj = 2 + p * NB + b
                bufB = b
                store(jj - 2, bufB).wait()
                gather(jj + 2, bufB).start()
                process(jj, (2 + b) % NB)

        for j in (chunks - 2, chunks - 1):
            process(j, j % NB)
        for b in range(NB):
            store(chunks - NB + b, b).wait()

    return emb(table, idx).transpose(2, 0, 1)
